# baseline (speedup 1.0000x reference)
import functools

import jax
import jax.numpy as jnp
from jax import lax
from jax.experimental import pallas as pl
from jax.experimental.pallas import tpu as pltpu
from jax.experimental.pallas import tpu_sc as plsc

D = 64
SI = 10
NBUF = 4


@functools.cache
def _make_gather(BATCH, S, W):
    info = plsc.get_sparse_core_info()
    nw = info.num_cores * info.num_subcores
    b_per_w = BATCH // nw
    cpb = S // SI
    n_chunks = b_per_w * cpb
    assert S % SI == 0 and BATCH % nw == 0 and n_chunks % NBUF == 0

    mesh = plsc.VectorSubcoreMesh(core_axis_name="c", subcore_axis_name="s")

    @functools.partial(
        pl.kernel,
        mesh=mesh,
        out_type=jax.ShapeDtypeStruct((BATCH, S, W, D), jnp.float32),
        scratch_types=[
            pltpu.VMEM((b_per_w, S, W), jnp.int32),
            pltpu.VMEM((NBUF, SI, W, D), jnp.float32),
        ]
        + [pltpu.SemaphoreType.DMA] * (2 * NBUF),
        compiler_params=pltpu.CompilerParams(use_tc_tiling_on_sc=False),
    )
    def emb(x_hbm, table_hbm, out_hbm, idx_all, rows, *sems):
        sem_g, sem_w = sems[:NBUF], sems[NBUF:]
        wid = lax.axis_index("s") * info.num_cores + lax.axis_index("c")
        b0 = wid * b_per_w

        pltpu.sync_copy(x_hbm.at[pl.ds(b0, b_per_w)], idx_all)

        def fire_gather(c, slot):
            bi = c // cpb
            si0 = (c % cpb) * SI
            for t in range(SI):
                pltpu.async_copy(
                    table_hbm.at[idx_all.at[bi, si0 + t]],
                    rows.at[slot].at[t],
                    sem_g[slot],
                )

        def wait_gather(slot):
            pltpu.make_async_copy(
                out_hbm.at[0].at[pl.ds(0, SI)], rows.at[slot], sem_g[slot]
            ).wait()

        def _write_copy(c, slot):
            bi = c // cpb
            si0 = (c % cpb) * SI
            return pltpu.make_async_copy(
                rows.at[slot],
                out_hbm.at[b0 + bi].at[pl.ds(si0, SI)],
                sem_w[slot],
            )

        def fire_write(c, slot):
            _write_copy(c, slot).start()

        def wait_write(c, slot):
            _write_copy(c, slot).wait()

        fire_gather(0, 0)
        fire_gather(1, 1)

        def group(t, carry):
            for b in range(NBUF):
                c = t * NBUF + b
                wait_gather(b)
                fire_write(c, b)
                s2 = (b + 2) % NBUF

                @pl.when(c + 2 < n_chunks)
                def _():
                    @pl.when(c >= 2)
                    def _():
                        wait_write(c - 2, s2)

                    fire_gather(c + 2, s2)

            return carry

        lax.fori_loop(0, n_chunks // NBUF, group, 0)

        for b in range(NBUF):
            wait_write(n_chunks - NBUF + b, b)

    return emb


def kernel(x, table):
    n_split = 4
    xi = x.astype(jnp.int32)
    bs = x.shape[0] // n_split
    emb = _make_gather(bs, *x.shape[1:])
    outs = [emb(xi[i * bs:(i + 1) * bs], table) for i in range(n_split)]
    return jnp.concatenate(outs, axis=0)


# speedup vs baseline: 2.3133x; 2.3133x over previous
import functools

import jax
import jax.numpy as jnp
from jax import lax
from jax.experimental import pallas as pl
from jax.experimental.pallas import tpu as pltpu
from jax.experimental.pallas import tpu_sc as plsc

D = 64
IDX_ROW = 128
CHUNK = 256
NBUF = 4
K = CHUNK // IDX_ROW


@functools.cache
def _make_gather(B):
    info = plsc.get_sparse_core_info()
    nw = info.num_cores * info.num_subcores
    assert B % (nw * CHUNK * NBUF) == 0
    b_per_w = B // nw
    n_chunks = b_per_w // CHUNK
    n_groups = n_chunks // NBUF
    n_idx_rows = b_per_w // IDX_ROW

    mesh = plsc.VectorSubcoreMesh(core_axis_name="c", subcore_axis_name="s")

    @functools.partial(
        pl.kernel,
        mesh=mesh,
        out_type=jax.ShapeDtypeStruct((B, 2 * D), jnp.float32),
        scratch_types=[
            pltpu.VMEM((n_idx_rows, IDX_ROW), jnp.int32),
            pltpu.VMEM((NBUF, CHUNK, D), jnp.float32),
        ]
        + [pltpu.SemaphoreType.DMA] * (2 * NBUF),
        compiler_params=pltpu.CompilerParams(use_tc_tiling_on_sc=False),
    )
    def emb(x_hbm, table_hbm, out_hbm, idx_all, rows, *sems):
        sem_g, sem_w = sems[:NBUF], sems[NBUF:]
        wid = lax.axis_index("s") * info.num_cores + lax.axis_index("c")
        row0 = wid * n_idx_rows
        out0 = wid * b_per_w

        pltpu.sync_copy(x_hbm.at[pl.ds(row0, n_idx_rows)], idx_all)

        def fire_gather(c, b):
            for j in range(K):
                pltpu.async_copy(
                    table_hbm.at[idx_all.at[c * K + j]],
                    rows.at[b].at[pl.ds(j * IDX_ROW, IDX_ROW)],
                    sem_g[b],
                )

        def wait_gather(b):
            pltpu.make_async_copy(
                table_hbm.at[pl.ds(0, CHUNK)], rows.at[b], sem_g[b]
            ).wait()

        def _write_copy(c, b):
            return pltpu.make_async_copy(
                rows.at[b],
                out_hbm.at[pl.ds(out0 + c * CHUNK, CHUNK), pl.ds(0, D)],
                sem_w[b],
            )

        def fire_write(c, b):
            _write_copy(c, b).start()

        def wait_write(c, b):
            _write_copy(c, b).wait()

        fire_gather(0, 0)
        fire_gather(1, 1)

        def group(t, carry):
            for b in range(NBUF):
                c = t * NBUF + b
                wait_gather(b)
                fire_write(c, b)
                s2 = (b + 2) % NBUF

                @pl.when(c + 2 < n_chunks)
                def _():
                    @pl.when(c >= 2)
                    def _():
                        wait_write(c - 2, s2)

                    fire_gather(c + 2, s2)

            return carry

        lax.fori_loop(0, n_groups, group, 0)

        for b in range(NBUF):
            wait_write(n_chunks - NBUF + b, b)

    return emb


def kernel(x, table):
    orig_shape = x.shape
    B = x.size
    x2d = x.reshape(B // IDX_ROW, IDX_ROW).astype(jnp.int32)
    out = _make_gather(B)(x2d, table)
    return out.reshape(*orig_shape, 2 * D)[..., :D]
